# baseline (speedup 1.0000x reference)
import functools

import jax
import jax.numpy as jnp
from jax.experimental import pallas as pl

_B, _C, _L, _V = 16, 32, 1024, 8192
_SEG = (1, 4, 16, 64, 256, 1024)
_BETA = 0.25
_PI = (0, 0, 1, 2, 3, 3)
_VC = 256


def _quant_body(q_ref, cbT_ref, emb_ref, h_ref, *, n_rows):
    nchunks = _V // _VC
    q = q_ref[...]

    def pass1(c, carry):
        m, idx = carry
        sc = jnp.dot(q, cbT_ref[:, pl.ds(c * _VC, _VC)],
                     preferred_element_type=jnp.float32)
        tmax = jnp.max(sc, axis=1, keepdims=True)
        lane = jax.lax.broadcasted_iota(jnp.int32, (n_rows, _VC), 1) + c * _VC
        targ = jnp.min(jnp.where(sc >= tmax, lane, _V), axis=1, keepdims=True)
        better = tmax > m
        return jnp.maximum(m, tmax), jnp.where(better, targ, idx)

    m0 = jnp.full((n_rows, 1), -jnp.inf, jnp.float32)
    i0 = jnp.zeros((n_rows, 1), jnp.int32)
    _, idx = jax.lax.fori_loop(0, nchunks, pass1, (m0, i0))

    def pass2(c, h):
        lane = jax.lax.broadcasted_iota(jnp.int32, (n_rows, _VC), 1) + c * _VC
        oh = (lane == idx).astype(jnp.float32)
        return h + jnp.dot(oh, emb_ref[pl.ds(c * _VC, _VC), :],
                           preferred_element_type=jnp.float32,
                           precision=jax.lax.Precision.HIGHEST)

    h_ref[...] = jax.lax.fori_loop(
        0, nchunks, pass2, jnp.zeros((n_rows, _C), jnp.float32))


def _quantize(q, cbT, emb_W):
    n = q.shape[0]
    rc = min(n, 2048)
    return pl.pallas_call(
        functools.partial(_quant_body, n_rows=rc),
        grid=(n // rc,),
        in_specs=[
            pl.BlockSpec((rc, _C), lambda i: (i, 0)),
            pl.BlockSpec((_C, _V), lambda i: (0, 0)),
            pl.BlockSpec((_V, _C), lambda i: (0, 0)),
        ],
        out_specs=pl.BlockSpec((rc, _C), lambda i: (i, 0)),
        out_shape=jax.ShapeDtypeStruct((n, _C), jnp.float32),
    )(q, cbT, emb_W)


def kernel(f_BCl, emb_W, phi_W, phi_b):
    f = f_BCl.transpose(0, 2, 1)
    norms = jnp.maximum(jnp.linalg.norm(emb_W, axis=1, keepdims=True), 1e-12)
    cbT = (emb_W / norms).T

    m_prev = phi_W[:, :, :, 0].transpose(0, 2, 1)
    m_cur = phi_W[:, :, :, 1].transpose(0, 2, 1)
    m_next = phi_W[:, :, :, 2].transpose(0, 2, 1)

    f_rest = f
    f_hat = jnp.zeros_like(f)
    loss = jnp.float32(0.0)
    zrow = jnp.zeros((_B, 1, _C), jnp.float32)
    for si, s in enumerate(_SEG):
        w = _L // s
        pooled = f_rest.reshape(_B, s, w, _C).mean(axis=2)
        q = pooled.reshape(_B * s, _C)
        q = q / jnp.maximum(jnp.linalg.norm(q, axis=1, keepdims=True), 1e-12)
        hseg = _quantize(q, cbT, emb_W).reshape(_B, s, _C)
        h = jnp.broadcast_to(hseg[:, :, None, :], (_B, s, w, _C))
        h = h.reshape(_B, _L, _C)
        pi = _PI[si]
        hp = jnp.concatenate([zrow, h[:, :-1]], axis=1)
        hn = jnp.concatenate([h[:, 1:], zrow], axis=1)
        conv = (hp @ m_prev[pi] + h @ m_cur[pi] + hn @ m_next[pi]
                + phi_b[pi][None, None])
        mixed = 0.5 * h + 0.5 * conv
        f_hat = f_hat + mixed
        f_rest = f_rest - mixed
        loss = loss + jnp.mean(f_rest ** 2)
    loss = (1.0 + _BETA) * loss / len(_SEG)
    return f_hat.transpose(0, 2, 1), loss


# speedup vs baseline: 1.8069x; 1.8069x over previous
import functools

import jax
import jax.numpy as jnp
from jax.experimental import pallas as pl
from jax.experimental.pallas import tpu as pltpu
from jax.experimental.pallas import tpu_sc as plsc

_B, _C, _L, _V = 16, 32, 1024, 8192
_SEG = (1, 4, 16, 64, 256, 1024)
_BETA = 0.25
_PI = (0, 0, 1, 2, 3, 3)
_VC = 256


def _quant_body(q_ref, cbT_ref, idx_ref, *, n_rows):
    nchunks = _V // _VC
    q = q_ref[...]

    def step(c, carry):
        m, idx = carry
        sc = jnp.dot(q, cbT_ref[:, pl.ds(c * _VC, _VC)],
                     preferred_element_type=jnp.float32)
        tmax = jnp.max(sc, axis=1, keepdims=True)
        lane = jax.lax.broadcasted_iota(jnp.int32, (n_rows, _VC), 1) + c * _VC
        targ = jnp.min(jnp.where(sc >= tmax, lane, _V), axis=1, keepdims=True)
        better = tmax > m
        return jnp.maximum(m, tmax), jnp.where(better, targ, idx)

    m0 = jnp.full((n_rows, 1), -jnp.inf, jnp.float32)
    i0 = jnp.zeros((n_rows, 1), jnp.int32)
    _, idx = jax.lax.fori_loop(0, nchunks, step, (m0, i0))
    idx_ref[...] = idx


def _quantize_idx(q, cbT):
    n = q.shape[0]
    rc = min(n, 2048)
    return pl.pallas_call(
        functools.partial(_quant_body, n_rows=rc),
        grid=(n // rc,),
        in_specs=[
            pl.BlockSpec((rc, _C), lambda i: (i, 0)),
            pl.BlockSpec((_C, _V), lambda i: (0, 0)),
        ],
        out_specs=pl.BlockSpec((rc, 1), lambda i: (i, 0)),
        out_shape=jax.ShapeDtypeStruct((n, 1), jnp.int32),
    )(q, cbT)


def _sc_gather_body(emb_hbm, idx_hbm, out_hbm, idx_v, rows_v, sem,
                    *, bpw, nw, chunk):
    wid = jax.lax.axis_index("s") * 2 + jax.lax.axis_index("c")

    @pl.when(wid < nw)
    def _():
        base = wid * bpw
        pltpu.sync_copy(idx_hbm.at[pl.ds(base, bpw)], idx_v)
        copies = []
        for k in range(bpw // chunk):
            copies.append(pltpu.async_copy(
                emb_hbm.at[idx_v.at[pl.ds(k * chunk, chunk)]],
                rows_v.at[pl.ds(k * chunk, chunk)], sem))
        for cp in copies:
            cp.wait()
        pltpu.sync_copy(rows_v, out_hbm.at[pl.ds(base, bpw)])


def _sc_gather(emb_pad, idx_flat):
    n = idx_flat.shape[0]
    nw = min(32, n // 8)
    bpw = n // nw
    chunk = min(128, bpw)
    mesh = plsc.VectorSubcoreMesh(core_axis_name="c", subcore_axis_name="s")
    f = pl.kernel(
        functools.partial(_sc_gather_body, bpw=bpw, nw=nw, chunk=chunk),
        mesh=mesh,
        out_type=jax.ShapeDtypeStruct((n, 128), jnp.float32),
        scratch_types=[
            pltpu.VMEM((bpw,), jnp.int32),
            pltpu.VMEM((bpw, 128), jnp.float32),
            pltpu.SemaphoreType.DMA,
        ],
    )
    return f(emb_pad, idx_flat)


def kernel(f_BCl, emb_W, phi_W, phi_b):
    f = f_BCl.transpose(0, 2, 1)
    norms = jnp.maximum(jnp.linalg.norm(emb_W, axis=1, keepdims=True), 1e-12)
    cbT = (emb_W / norms).T
    emb_pad = jnp.pad(emb_W, ((0, 0), (0, 128 - _C)))

    m_prev = phi_W[:, :, :, 0].transpose(0, 2, 1)
    m_cur = phi_W[:, :, :, 1].transpose(0, 2, 1)
    m_next = phi_W[:, :, :, 2].transpose(0, 2, 1)

    f_rest = f
    f_hat = jnp.zeros_like(f)
    loss = jnp.float32(0.0)
    zrow = jnp.zeros((_B, 1, _C), jnp.float32)
    for si, s in enumerate(_SEG):
        w = _L // s
        pooled = f_rest.reshape(_B, s, w, _C).mean(axis=2)
        q = pooled.reshape(_B * s, _C)
        q = q / jnp.maximum(jnp.linalg.norm(q, axis=1, keepdims=True), 1e-12)
        idx = _quantize_idx(q, cbT).reshape(_B * s)
        hseg = _sc_gather(emb_pad, idx)[:, :_C].reshape(_B, s, _C)
        h = jnp.broadcast_to(hseg[:, :, None, :], (_B, s, w, _C))
        h = h.reshape(_B, _L, _C)
        pi = _PI[si]
        hp = jnp.concatenate([zrow, h[:, :-1]], axis=1)
        hn = jnp.concatenate([h[:, 1:], zrow], axis=1)
        conv = (hp @ m_prev[pi] + h @ m_cur[pi] + hn @ m_next[pi]
                + phi_b[pi][None, None])
        mixed = 0.5 * h + 0.5 * conv
        f_hat = f_hat + mixed
        f_rest = f_rest - mixed
        loss = loss + jnp.mean(f_rest ** 2)
    loss = (1.0 + _BETA) * loss / len(_SEG)
    return f_hat.transpose(0, 2, 1), loss
